# Initial kernel scaffold; baseline (speedup 1.0000x reference)
import jax
import jax.numpy as jnp
from jax import lax
from jax.experimental import pallas as pl
from jax.experimental.pallas import tpu as pltpu
from jax.experimental.pallas import tpu_sc as plsc

B = 1024
S = 200
H = 128
HALF = 100
NC = 2
NS = 16
NW = NC * NS
ROWS_PER_W = B // NW
LANES = 16
KCH = H // LANES


def _body(ids_h, cidx_h, pos_h, typ_h, ten_h, word_h, out_h,
          pos_v, combo_v, typ_v, acc_v, idx_a, idx_b, cidx_v, gsem):
    c = lax.axis_index("c")
    s = lax.axis_index("s")
    wid = s * NC + c

    pltpu.sync_copy(pos_h, pos_v)
    pltpu.sync_copy(typ_h, typ_v)
    pltpu.sync_copy(ten_h, combo_v.at[pl.ds(0, HALF)])
    pltpu.sync_copy(ten_h, combo_v.at[pl.ds(HALF, HALF)])

    def build(t, carry):
        for half in range(2):
            for k in range(KCH):
                sl = pl.ds(k * LANES, LANES)
                combo_v[half * HALF + t, sl] = (
                    combo_v[half * HALF + t, sl] + typ_v[half, sl])
        return carry
    lax.fori_loop(0, HALF, build, 0)

    def row(r, carry):
        b = wid * ROWS_PER_W + r
        pltpu.sync_copy(ids_h.at[b, 0], idx_a)
        pltpu.sync_copy(ids_h.at[b, 1], idx_b)
        pltpu.sync_copy(cidx_h.at[b], cidx_v)
        ga = pltpu.async_copy(word_h.at[idx_a], acc_v.at[0], gsem)
        gb = pltpu.async_copy(word_h.at[idx_b], acc_v.at[1], gsem)
        ga.wait()
        gb.wait()

        def tok(t, inner):
            for half in range(2):
                ct = cidx_v[half, t]
                for k in range(KCH):
                    sl = pl.ds(k * LANES, LANES)
                    acc_v[half, t, sl] = (
                        acc_v[half, t, sl]
                        + pos_v[half, t, sl]
                        + combo_v[ct, sl])
            return inner
        lax.fori_loop(0, HALF, tok, 0)

        pltpu.sync_copy(acc_v, out_h.at[b])
        return carry
    lax.fori_loop(0, ROWS_PER_W, row, 0)


@jax.jit
def _run(ids, cidx, pos, typ, ten, word):
    mesh = plsc.VectorSubcoreMesh(core_axis_name="c", subcore_axis_name="s")
    return pl.kernel(
        _body,
        out_type=jax.ShapeDtypeStruct((B, 2, HALF, H), jnp.float32),
        mesh=mesh,
        scratch_types=[
            pltpu.VMEM((2, HALF, H), jnp.float32),
            pltpu.VMEM((2 * HALF, H), jnp.float32),
            pltpu.VMEM((2, H), jnp.float32),
            pltpu.VMEM((2, HALF, H), jnp.float32),
            pltpu.VMEM((HALF,), jnp.int32),
            pltpu.VMEM((HALF,), jnp.int32),
            pltpu.VMEM((2, HALF), jnp.int32),
            pltpu.SemaphoreType.DMA,
        ],
    )(ids, cidx, pos, typ, ten, word)


def kernel(input_ids, token_type_ids, tenant_ids, W_word, W_pos, W_type, W_tenant):
    ids = input_ids.astype(jnp.int32).reshape(B, 2, HALF)
    cidx = (token_type_ids.astype(jnp.int32) * 100
            + tenant_ids.astype(jnp.int32)).reshape(B, 2, HALF)
    pos = W_pos[:S].reshape(2, HALF, H)
    out = _run(ids, cidx, pos, W_type, W_tenant, W_word)
    return out.reshape(B, S, H)


# speedup vs baseline: 6.9485x; 6.9485x over previous
import jax
import jax.numpy as jnp
from jax import lax
from jax.experimental import pallas as pl
from jax.experimental.pallas import tpu as pltpu
from jax.experimental.pallas import tpu_sc as plsc

B = 1024
S = 200
H = 128
SPLIT_A = 104
SPLIT_B = S - SPLIT_A
NC = 2
NS = 16
NW = NC * NS
ROWS_PER_W = B // NW
LANES = 16
KCH = H // LANES
NQ = S // LANES
TAIL = S - NQ * LANES
TEN_PAD = 104


def _body(ids_h, cidx_h, pos_h, typ_h, ten_h, word_h, out_h,
          pos_v, combo_v, typ_v, ten_v, acc_v, idx_a, idx_b, cidx_v, gsem):
    c = lax.axis_index("c")
    s = lax.axis_index("s")
    wid = s * NC + c

    pltpu.sync_copy(pos_h, pos_v)
    pltpu.sync_copy(typ_h, typ_v)
    pltpu.sync_copy(ten_h, ten_v)

    def build(t, carry):
        for half in range(2):
            for k in range(KCH):
                sl = pl.ds(k * LANES, LANES)
                combo_v[half * 100 + t, sl] = (
                    ten_v[t, sl] + typ_v[pl.ds(half * H + k * LANES, LANES)])
        return carry
    lax.fori_loop(0, 100, build, 0)

    def do_token(t, ct):
        for k in range(KCH):
            sl = pl.ds(k * LANES, LANES)
            acc_v[t, sl] = acc_v[t, sl] + pos_v[t, sl] + combo_v[ct, sl]

    def row(r, carry):
        b = wid * ROWS_PER_W + r
        base = b * S
        pltpu.sync_copy(ids_h.at[pl.ds(base, SPLIT_A)], idx_a)
        pltpu.sync_copy(ids_h.at[pl.ds(base + SPLIT_A, SPLIT_B)], idx_b)
        pltpu.sync_copy(cidx_h.at[pl.ds(base, S)], cidx_v.at[pl.ds(0, S)])
        ga = pltpu.async_copy(word_h.at[idx_a], acc_v.at[pl.ds(0, SPLIT_A)],
                              gsem)
        gb = pltpu.async_copy(word_h.at[idx_b],
                              acc_v.at[pl.ds(SPLIT_A, SPLIT_B)], gsem)
        ga.wait()
        gb.wait()

        def group(q, inner):
            t0 = q * LANES
            chunk = cidx_v[pl.ds(t0, LANES)]
            for i in range(LANES):
                do_token(t0 + i, chunk[i])
            return inner
        lax.fori_loop(0, NQ, group, 0)

        tail_chunk = cidx_v[pl.ds(NQ * LANES, LANES)]
        for i in range(TAIL):
            do_token(NQ * LANES + i, tail_chunk[i])

        pltpu.sync_copy(acc_v, out_h.at[b])
        return carry
    lax.fori_loop(0, ROWS_PER_W, row, 0)


@jax.jit
def _run(ids, cidx, pos, typ, ten, word):
    mesh = plsc.VectorSubcoreMesh(core_axis_name="c", subcore_axis_name="s")
    return pl.kernel(
        _body,
        out_type=jax.ShapeDtypeStruct((B, S, H), jnp.float32),
        mesh=mesh,
        scratch_types=[
            pltpu.VMEM((S, H), jnp.float32),
            pltpu.VMEM((S, H), jnp.float32),
            pltpu.VMEM((2 * H,), jnp.float32),
            pltpu.VMEM((TEN_PAD, H), jnp.float32),
            pltpu.VMEM((S, H), jnp.float32),
            pltpu.VMEM((SPLIT_A,), jnp.int32),
            pltpu.VMEM((SPLIT_B,), jnp.int32),
            pltpu.VMEM(((NQ + 1) * LANES,), jnp.int32),
            pltpu.SemaphoreType.DMA,
        ],
    )(ids, cidx, pos, typ, ten, word)


def kernel(input_ids, token_type_ids, tenant_ids, W_word, W_pos, W_type, W_tenant):
    ids = input_ids.astype(jnp.int32).reshape(B * S)
    cidx = (token_type_ids.astype(jnp.int32) * 100
            + tenant_ids.astype(jnp.int32)).reshape(B * S)
    pos = W_pos[:S]
    typ = W_type.reshape(2 * H)
    ten = jnp.pad(W_tenant, ((0, TEN_PAD - W_tenant.shape[0]), (0, 0)))
    return _run(ids, cidx, pos, typ, ten, W_word)
